# initial kernel scaffold (unmeasured)
import jax
import jax.numpy as jnp
from jax import lax
from jax.experimental import pallas as pl
from jax.experimental.pallas import tpu as pltpu

N_DEV = 16
TAIL = 256


def kernel(x, A, B, C):
    Bb, S, D = x.shape
    N = A.shape[1]

    def body(x_ref, A_ref, B_ref, C_ref, y_ref, lbuf, pbuf, send_sem, recv_sem):
        my = lax.axis_index("i")
        right = lax.rem(my + 1, N_DEV)

        dAT = jnp.exp(A_ref[...]).T[None]
        BT = jnp.transpose(B_ref[...], (0, 2, 1))
        CT = jnp.transpose(C_ref[...], (0, 2, 1))

        def tail_step(t, h):
            xt = pl.load(x_ref, (slice(None), pl.ds(t, 1), slice(None)))
            bt = lax.dynamic_slice(BT, (0, 0, t), (Bb, N, 1))
            return h * dAT + xt * bt

        l = lax.fori_loop(
            S - TAIL, S, tail_step, jnp.zeros((Bb, N, D), jnp.float32)
        )
        lbuf[...] = l

        rdma = pltpu.make_async_remote_copy(
            src_ref=lbuf,
            dst_ref=pbuf,
            send_sem=send_sem,
            recv_sem=recv_sem,
            device_id=(right,),
            device_id_type=pl.DeviceIdType.MESH,
        )
        rdma.start()
        rdma.wait()

        mask = (my != 0).astype(jnp.float32)
        h0 = pbuf[...] * mask

        def step(t, h):
            xt = pl.load(x_ref, (slice(None), pl.ds(t, 1), slice(None)))
            bt = lax.dynamic_slice(BT, (0, 0, t), (Bb, N, 1))
            ct = lax.dynamic_slice(CT, (0, 0, t), (Bb, N, 1))
            h = h * dAT + xt * bt
            yt = jnp.sum(h * ct, axis=1)
            pl.store(
                y_ref,
                (slice(None), pl.ds(t, 1), slice(None)),
                yt[:, None, :],
            )
            return h

        lax.fori_loop(0, S, step, h0)

    return pl.pallas_call(
        body,
        out_shape=jax.ShapeDtypeStruct((Bb, S, D), jnp.float32),
        in_specs=[pl.BlockSpec(memory_space=pltpu.VMEM)] * 4,
        out_specs=pl.BlockSpec(memory_space=pltpu.VMEM),
        scratch_shapes=[
            pltpu.VMEM((Bb, N, D), jnp.float32),
            pltpu.VMEM((Bb, N, D), jnp.float32),
            pltpu.SemaphoreType.DMA,
            pltpu.SemaphoreType.DMA,
        ],
    )(x, A, B, C)


# baseline (device time: 393426 ns/iter reference)
import jax
import jax.numpy as jnp
from jax import lax
from jax.experimental import pallas as pl
from jax.experimental.pallas import tpu as pltpu

N_DEV = 16
TAIL = 256


def kernel(x, A, B, C):
    Bb, S, D = x.shape
    N = A.shape[1]

    def body(
        x_ref, A_ref, B_ref, C_ref, y_ref,
        lbuf, pbuf, send_sem, recv_sem,
    ):
        my = lax.axis_index("i")
        right = lax.rem(my + 1, N_DEV)

        dAT = jnp.exp(A_ref[...]).T[None]

        def tail_step(t, h):
            xt = x_ref[:, pl.ds(t, 1), :]
            bt = jnp.swapaxes(B_ref[:, pl.ds(t, 1), :], 1, 2)
            return h * dAT + xt * bt

        l = lax.fori_loop(
            S - TAIL, S, tail_step, jnp.zeros((Bb, N, D), jnp.float32)
        )
        lbuf[...] = l

        rdma = pltpu.make_async_remote_copy(
            src_ref=lbuf,
            dst_ref=pbuf,
            send_sem=send_sem,
            recv_sem=recv_sem,
            device_id=(right,),
            device_id_type=pl.DeviceIdType.MESH,
        )
        rdma.start()
        rdma.wait()

        mask = (my != 0).astype(jnp.float32)
        h0 = pbuf[...] * mask

        def step(t, h):
            xt = x_ref[:, pl.ds(t, 1), :]
            bt = jnp.swapaxes(B_ref[:, pl.ds(t, 1), :], 1, 2)
            ct = jnp.swapaxes(C_ref[:, pl.ds(t, 1), :], 1, 2)
            h = h * dAT + xt * bt
            yt = jnp.sum(h * ct, axis=1)
            y_ref[:, pl.ds(t, 1), :] = yt[:, None, :]
            return h

        lax.fori_loop(0, S, step, h0)

    return pl.pallas_call(
        body,
        out_shape=jax.ShapeDtypeStruct((Bb, S, D), jnp.float32),
        in_specs=[pl.BlockSpec(memory_space=pltpu.VMEM)] * 4,
        out_specs=pl.BlockSpec(memory_space=pltpu.VMEM),
        scratch_shapes=[
            pltpu.VMEM((Bb, N, D), jnp.float32),
            pltpu.VMEM((Bb, N, D), jnp.float32),
            pltpu.SemaphoreType.DMA,
            pltpu.SemaphoreType.DMA,
        ],
    )(x, A, B, C)
